# Initial kernel scaffold; baseline (speedup 1.0000x reference)
import jax
import jax.numpy as jnp
from jax.experimental import pallas as pl


def kernel(context_embeds, params, tokens):
    raise NotImplementedError("write your pallas kernel here")


# speedup vs baseline: 4.1497x; 4.1497x over previous
import functools

import jax
import jax.numpy as jnp
from jax import lax
from jax.experimental import pallas as pl
from jax.experimental.pallas import tpu as pltpu
from jax.experimental.pallas import tpu_sc as plsc

N_INF = -1e10
L, B, D, V = 512, 128, 128, 128
TOK_PAD, TOK_START, TOK_CONST, TOK_SUB, TOK_STOP = 0, 1, 2, 3, 4
D4 = 4 * D
D2 = 2 * D
NHEAD = 4
TL = 8


def _scan_body(reverse, tokens_ref, ctx_ref, wc_ref, b0_ref, emb_ref, wi_ref,
               bi_ref, wh_ref, hs_ref, tf_ref, h_ref, c_ref):
    j = pl.program_id(0)

    @pl.when(j == 0)
    def _init():
        tf_ref[...] = (
            jnp.dot(emb_ref[...], wi_ref[...], preferred_element_type=jnp.float32)
            + bi_ref[...])
        hc = jnp.tanh(
            jnp.dot(ctx_ref[...], wc_ref[...], preferred_element_type=jnp.float32)
            + b0_ref[...])
        h_ref[...] = hc[:, :D]
        c_ref[...] = hc[:, D:]

    t = (L - 1 - j) if reverse else j
    tok = tokens_ref[pl.ds(t, 1), :]
    oh = (lax.broadcasted_iota(jnp.int32, (V, B), 0) == tok).astype(jnp.float32)
    x = lax.dot_general(oh, tf_ref[...], (((0,), (0,)), ((), ())),
                        preferred_element_type=jnp.float32)
    gates = x + jnp.dot(h_ref[...], wh_ref[...],
                        preferred_element_type=jnp.float32)
    i_g = jax.nn.sigmoid(gates[:, :D])
    f_g = jax.nn.sigmoid(gates[:, D:2 * D])
    g_g = jnp.tanh(gates[:, 2 * D:3 * D])
    o_g = jax.nn.sigmoid(gates[:, 3 * D:])
    c = f_g * c_ref[...] + i_g * g_g
    h = o_g * jnp.tanh(c)
    h_ref[...] = h
    c_ref[...] = c
    hs_ref[0, :, :] = h


def _lstm_scan(reverse, tokens, ctx, wc, b0, emb, wi, bi, wh):
    out_map = (lambda j: (L - 1 - j, 0, 0)) if reverse else (lambda j: (j, 0, 0))
    return pl.pallas_call(
        functools.partial(_scan_body, reverse),
        grid=(L,),
        in_specs=[
            pl.BlockSpec((L, B), lambda j: (0, 0)),
            pl.BlockSpec((B, D), lambda j: (0, 0)),
            pl.BlockSpec((D, D2), lambda j: (0, 0)),
            pl.BlockSpec((1, D2), lambda j: (0, 0)),
            pl.BlockSpec((V, D), lambda j: (0, 0)),
            pl.BlockSpec((D, D4), lambda j: (0, 0)),
            pl.BlockSpec((1, D4), lambda j: (0, 0)),
            pl.BlockSpec((D, D4), lambda j: (0, 0)),
        ],
        out_specs=pl.BlockSpec((1, B, D), out_map),
        out_shape=jax.ShapeDtypeStruct((L, B, D), jnp.float32),
        scratch_shapes=[
            pltpu.VMEM((V, D4), jnp.float32),
            pltpu.VMEM((B, D), jnp.float32),
            pltpu.VMEM((B, D), jnp.float32),
        ],
        compiler_params=pltpu.CompilerParams(
            dimension_semantics=("arbitrary",)),
    )(tokens, ctx, wc, b0, emb, wi, bi, wh)


def _heads_body(hf_ref, hb_ref, w1_ref, b1_ref, w2_ref, b2_ref, sc_ref):
    hf = hf_ref[...].reshape(TL * B, D)
    hb = hb_ref[...].reshape(TL * B, D)
    out2 = jnp.concatenate([hf, hb], axis=1)
    hid = jnp.dot(out2, w1_ref[...], preferred_element_type=jnp.float32)
    hid = jnp.maximum(hid + b1_ref[...], 0.0)
    st = lax.dot_general(w2_ref[...], hid, (((0,), (1,)), ((), ())),
                         preferred_element_type=jnp.float32)
    st = st + b2_ref[...]
    sc_ref[...] = st.reshape(NHEAD, TL, B)


def _heads(hs_f, hs_b, w1, b1, w2, b2):
    return pl.pallas_call(
        _heads_body,
        grid=(L // TL,),
        in_specs=[
            pl.BlockSpec((TL, B, D), lambda j: (j, 0, 0)),
            pl.BlockSpec((TL, B, D), lambda j: (j, 0, 0)),
            pl.BlockSpec((D2, NHEAD * D2), lambda j: (0, 0)),
            pl.BlockSpec((1, NHEAD * D2), lambda j: (0, 0)),
            pl.BlockSpec((NHEAD * D2, NHEAD), lambda j: (0, 0)),
            pl.BlockSpec((NHEAD, 1), lambda j: (0, 0)),
        ],
        out_specs=pl.BlockSpec((NHEAD, TL, B), lambda j: (0, j, 0)),
        out_shape=jax.ShapeDtypeStruct((NHEAD, L, B), jnp.float32),
        compiler_params=pltpu.CompilerParams(
            dimension_semantics=("arbitrary",)),
    )(hs_f, hs_b, w1, b1, w2, b2)


def _assemble_body(sc_ref, tok_ref, ll_ref, tgt_ref, gidx_ref):
    tok = tok_ref[...]
    mod_s = sc_ref[0]
    del_s = sc_ref[1]
    ins_s = sc_ref[2]
    stop_s = sc_ref[3]
    expr = (tok == TOK_CONST) | (tok == TOK_SUB)
    zf = jnp.zeros((1, B), dtype=jnp.float32)
    expr_f = expr.astype(jnp.float32)
    expr_sh = jnp.concatenate([zf, expr_f[:-1]], axis=0) != 0.0
    del_sh = jnp.concatenate([zf, del_s[:-1]], axis=0)
    score = jnp.full((L, B), N_INF, dtype=jnp.float32)
    score = jnp.where(expr, mod_s, score)
    score = jnp.where(expr_sh, del_sh, score)
    score = jnp.where(tok == TOK_START, ins_s, score)
    score = jnp.where(tok == TOK_STOP, stop_s, score)
    m = jnp.max(score, axis=0, keepdims=True)
    z = jnp.log(jnp.sum(jnp.exp(score - m), axis=0, keepdims=True))
    ll_ref[...] = -z
    iot = lax.broadcasted_iota(jnp.int32, (L, B), 0)
    cand = jnp.where(score == m, iot, L)
    tgt = jnp.min(cand, axis=0, keepdims=True)
    tgt_ref[...] = tgt
    gidx_ref[...] = tgt * B + lax.broadcasted_iota(jnp.int32, (1, B), 1)


def _assemble(scores, tokens):
    return pl.pallas_call(
        _assemble_body,
        out_shape=[
            jax.ShapeDtypeStruct((1, B), jnp.float32),
            jax.ShapeDtypeStruct((1, B), jnp.int32),
            jax.ShapeDtypeStruct((1, B), jnp.int32),
        ],
    )(scores, tokens)


_ROWS_PER_W = 16
_NW_ACT = B // _ROWS_PER_W


def _gather_sc_body(hsf_hbm, hsb_hbm, gidx_hbm, updf_hbm, updb_hbm,
                    idx_v, rf_v, rb_v, sem):
    wid = lax.axis_index("s") * 2 + lax.axis_index("c")

    @pl.when(wid < _NW_ACT)
    def _():
        base = wid * _ROWS_PER_W
        pltpu.sync_copy(gidx_hbm.at[pl.ds(base, _ROWS_PER_W)], idx_v)
        pltpu.async_copy(hsf_hbm.at[idx_v], rf_v, sem).wait()
        pltpu.async_copy(hsb_hbm.at[idx_v], rb_v, sem).wait()
        pltpu.sync_copy(rf_v, updf_hbm.at[pl.ds(base, _ROWS_PER_W)])
        pltpu.sync_copy(rb_v, updb_hbm.at[pl.ds(base, _ROWS_PER_W)])


@functools.cache
def _gather_sc_kernel():
    return pl.kernel(
        _gather_sc_body,
        out_type=[
            jax.ShapeDtypeStruct((B, D), jnp.float32),
            jax.ShapeDtypeStruct((B, D), jnp.float32),
        ],
        mesh=plsc.VectorSubcoreMesh(core_axis_name="c", subcore_axis_name="s"),
        scratch_types=[
            pltpu.VMEM((_ROWS_PER_W,), jnp.int32),
            pltpu.VMEM((_ROWS_PER_W, D), jnp.float32),
            pltpu.VMEM((_ROWS_PER_W, D), jnp.float32),
            pltpu.SemaphoreType.DMA,
        ],
    )


def kernel(context_embeds, params, tokens):
    p = params

    wc_f = jnp.concatenate([p['Wch'][:, :D], p['Wcc'][:, :D]], axis=1)
    wc_b = jnp.concatenate([p['Wch'][:, D:], p['Wcc'][:, D:]], axis=1)
    b0_f = jnp.concatenate([p['bch'][:D], p['bcc'][:D]]).reshape(1, D2)
    b0_b = jnp.concatenate([p['bch'][D:], p['bcc'][D:]]).reshape(1, D2)
    bi_f = p['b_f'].reshape(1, D4)
    bi_b = p['b_b'].reshape(1, D4)
    names = ['mod', 'dele', 'ins', 'stop']
    w1 = jnp.concatenate([p[nm + '_W1'] for nm in names], axis=1)
    b1 = jnp.concatenate([p[nm + '_b1'] for nm in names]).reshape(1, NHEAD * D2)
    w2 = jnp.zeros((NHEAD * D2, NHEAD), jnp.float32)
    for k, nm in enumerate(names):
        w2 = w2.at[k * D2:(k + 1) * D2, k].set(p[nm + '_W2'][:, 0])
    b2 = jnp.stack([p[nm + '_b2'][0] for nm in names]).reshape(NHEAD, 1)

    hs_f = _lstm_scan(False, tokens, context_embeds, wc_f, b0_f,
                      p['emb'], p['Wi_f'], bi_f, p['Wh_f'])
    hs_b = _lstm_scan(True, tokens, context_embeds, wc_b, b0_b,
                      p['emb'], p['Wi_b'], bi_b, p['Wh_b'])
    scores = _heads(hs_f, hs_b, w1, b1, w2, b2)
    ll, tgt, gidx = _assemble(scores, tokens)
    updf, updb = _gather_sc_kernel()(hs_f.reshape(L * B, D),
                                     hs_b.reshape(L * B, D), gidx.reshape(B))
    update_embed = jnp.concatenate([updf, updb], axis=1)
    return ll.reshape(B, 1), tgt.reshape(B), update_embed


# speedup vs baseline: 6.4946x; 1.5651x over previous
import functools

import jax
import jax.numpy as jnp
from jax import lax
from jax.experimental import pallas as pl
from jax.experimental.pallas import tpu as pltpu
from jax.experimental.pallas import tpu_sc as plsc

N_INF = -1e10
L, B, D, V = 512, 128, 128, 128
TOK_PAD, TOK_START, TOK_CONST, TOK_SUB, TOK_STOP = 0, 1, 2, 3, 4
D4 = 4 * D
D2 = 2 * D
NHEAD = 4
TL = 8


def _sig(x):
    return 0.5 * jnp.tanh(0.5 * x) + 0.5


def _prep_body(emb_ref, wif_ref, bif_ref, wib_ref, bib_ref, ctx_ref,
               wcf_ref, b0f_ref, wcb_ref, b0b_ref,
               tff_ref, tfb_ref, hcf_ref, hcb_ref):
    emb = emb_ref[...]
    tff_ref[...] = jnp.dot(emb, wif_ref[...],
                           preferred_element_type=jnp.float32) + bif_ref[...]
    tfb_ref[...] = jnp.dot(emb, wib_ref[...],
                           preferred_element_type=jnp.float32) + bib_ref[...]
    ctx = ctx_ref[...]
    hcf_ref[...] = jnp.tanh(jnp.dot(ctx, wcf_ref[...],
                                    preferred_element_type=jnp.float32)
                            + b0f_ref[...])
    hcb_ref[...] = jnp.tanh(jnp.dot(ctx, wcb_ref[...],
                                    preferred_element_type=jnp.float32)
                            + b0b_ref[...])


def _prep(emb, wif, bif, wib, bib, ctx, wcf, b0f, wcb, b0b):
    return pl.pallas_call(
        _prep_body,
        out_shape=[
            jax.ShapeDtypeStruct((V, D4), jnp.float32),
            jax.ShapeDtypeStruct((V, D4), jnp.float32),
            jax.ShapeDtypeStruct((B, D2), jnp.float32),
            jax.ShapeDtypeStruct((B, D2), jnp.float32),
        ],
    )(emb, wif, bif, wib, bib, ctx, wcf, b0f, wcb, b0b)


def _cell(tok, tf_ref, wh_ref, h, c):
    oh = (lax.broadcasted_iota(jnp.int32, (V, B), 0) == tok).astype(jnp.float32)
    x = lax.dot_general(oh, tf_ref[...], (((0,), (0,)), ((), ())),
                        preferred_element_type=jnp.float32)
    gates = x + jnp.dot(h, wh_ref[...], preferred_element_type=jnp.float32)
    i_g = _sig(gates[:, :D])
    f_g = _sig(gates[:, D:2 * D])
    g_g = jnp.tanh(gates[:, 2 * D:3 * D])
    o_g = _sig(gates[:, 3 * D:])
    c_n = f_g * c + i_g * g_g
    h_n = o_g * jnp.tanh(c_n)
    return h_n, c_n


def _scan_body(tokens_ref, tff_ref, whf_ref, tfb_ref, whb_ref,
               hcf_ref, hcb_ref, hsf_ref, hsb_ref,
               hf_s, cf_s, hb_s, cb_s):
    j = pl.program_id(0)

    @pl.when(j == 0)
    def _init():
        hf_s[...] = hcf_ref[:, :D]
        cf_s[...] = hcf_ref[:, D:]
        hb_s[...] = hcb_ref[:, :D]
        cb_s[...] = hcb_ref[:, D:]

    tok_f = tokens_ref[pl.ds(j, 1), :]
    tok_b = tokens_ref[pl.ds(L - 1 - j, 1), :]
    hf, cf = _cell(tok_f, tff_ref, whf_ref, hf_s[...], cf_s[...])
    hb, cb = _cell(tok_b, tfb_ref, whb_ref, hb_s[...], cb_s[...])
    hf_s[...] = hf
    cf_s[...] = cf
    hb_s[...] = hb
    cb_s[...] = cb
    hsf_ref[0, :, :] = hf
    hsb_ref[0, :, :] = hb


def _lstm_scan(tokens, tff, whf, tfb, whb, hcf, hcb):
    cparams = pltpu.CompilerParams(dimension_semantics=("arbitrary",))
    return pl.pallas_call(
        _scan_body,
        grid=(L,),
        in_specs=[
            pl.BlockSpec((L, B), lambda j: (0, 0)),
            pl.BlockSpec((V, D4), lambda j: (0, 0)),
            pl.BlockSpec((D, D4), lambda j: (0, 0)),
            pl.BlockSpec((V, D4), lambda j: (0, 0)),
            pl.BlockSpec((D, D4), lambda j: (0, 0)),
            pl.BlockSpec((B, D2), lambda j: (0, 0)),
            pl.BlockSpec((B, D2), lambda j: (0, 0)),
        ],
        out_specs=[
            pl.BlockSpec((1, B, D), lambda j: (j, 0, 0)),
            pl.BlockSpec((1, B, D), lambda j: (L - 1 - j, 0, 0)),
        ],
        out_shape=[
            jax.ShapeDtypeStruct((L, B, D), jnp.float32),
            jax.ShapeDtypeStruct((L, B, D), jnp.float32),
        ],
        scratch_shapes=[pltpu.VMEM((B, D), jnp.float32) for _ in range(4)],
        compiler_params=cparams,
    )(tokens, tff, whf, tfb, whb, hcf, hcb)


def _heads_body(hf_ref, hb_ref, w1_ref, b1_ref, w2_ref, b2_ref, tok_ref,
                ll_ref, tgt_ref, gidx_ref, sc_s):
    j = pl.program_id(0)
    hf = hf_ref[...].reshape(TL * B, D)
    hb = hb_ref[...].reshape(TL * B, D)
    out2 = jnp.concatenate([hf, hb], axis=1)
    hid = jnp.dot(out2, w1_ref[...], preferred_element_type=jnp.float32)
    hid = jnp.maximum(hid + b1_ref[...], 0.0)
    st = lax.dot_general(w2_ref[...], hid, (((0,), (1,)), ((), ())),
                         preferred_element_type=jnp.float32)
    st = st + b2_ref[...]
    sc_s[:, pl.ds(j * TL, TL), :] = st.reshape(NHEAD, TL, B)

    @pl.when(j == L // TL - 1)
    def _assemble():
        tok = tok_ref[...]
        mod_s = sc_s[0]
        del_s = sc_s[1]
        ins_s = sc_s[2]
        stop_s = sc_s[3]
        expr = (tok == TOK_CONST) | (tok == TOK_SUB)
        zf = jnp.zeros((1, B), dtype=jnp.float32)
        expr_f = expr.astype(jnp.float32)
        expr_sh = jnp.concatenate([zf, expr_f[:-1]], axis=0) != 0.0
        del_sh = jnp.concatenate([zf, del_s[:-1]], axis=0)
        score = jnp.full((L, B), N_INF, dtype=jnp.float32)
        score = jnp.where(expr, mod_s, score)
        score = jnp.where(expr_sh, del_sh, score)
        score = jnp.where(tok == TOK_START, ins_s, score)
        score = jnp.where(tok == TOK_STOP, stop_s, score)
        m = jnp.max(score, axis=0, keepdims=True)
        z = jnp.log(jnp.sum(jnp.exp(score - m), axis=0, keepdims=True))
        ll_ref[...] = -z
        iot = lax.broadcasted_iota(jnp.int32, (L, B), 0)
        cand = jnp.where(score == m, iot, L)
        tgt = jnp.min(cand, axis=0, keepdims=True)
        tgt_ref[...] = tgt
        gidx_ref[...] = tgt * B + lax.broadcasted_iota(jnp.int32, (1, B), 1)


def _heads(hs_f, hs_b, w1, b1, w2, b2, tokens):
    return pl.pallas_call(
        _heads_body,
        grid=(L // TL,),
        in_specs=[
            pl.BlockSpec((TL, B, D), lambda j: (j, 0, 0)),
            pl.BlockSpec((TL, B, D), lambda j: (j, 0, 0)),
            pl.BlockSpec((D2, NHEAD * D2), lambda j: (0, 0)),
            pl.BlockSpec((1, NHEAD * D2), lambda j: (0, 0)),
            pl.BlockSpec((NHEAD * D2, NHEAD), lambda j: (0, 0)),
            pl.BlockSpec((NHEAD, 1), lambda j: (0, 0)),
            pl.BlockSpec((L, B), lambda j: (0, 0)),
        ],
        out_specs=[
            pl.BlockSpec((1, B), lambda j: (0, 0)),
            pl.BlockSpec((1, B), lambda j: (0, 0)),
            pl.BlockSpec((1, B), lambda j: (0, 0)),
        ],
        out_shape=[
            jax.ShapeDtypeStruct((1, B), jnp.float32),
            jax.ShapeDtypeStruct((1, B), jnp.int32),
            jax.ShapeDtypeStruct((1, B), jnp.int32),
        ],
        scratch_shapes=[pltpu.VMEM((NHEAD, L, B), jnp.float32)],
        compiler_params=pltpu.CompilerParams(
            dimension_semantics=("arbitrary",)),
    )(hs_f, hs_b, w1, b1, w2, b2, tokens)


_ROWS_PER_W = 16
_NW_ACT = B // _ROWS_PER_W


def _gather_sc_body(hsf_hbm, hsb_hbm, gidx_hbm, updf_hbm, updb_hbm,
                    idx_v, rf_v, rb_v, sem):
    wid = lax.axis_index("s") * 2 + lax.axis_index("c")

    @pl.when(wid < _NW_ACT)
    def _():
        base = wid * _ROWS_PER_W
        pltpu.sync_copy(gidx_hbm.at[pl.ds(base, _ROWS_PER_W)], idx_v)
        pltpu.async_copy(hsf_hbm.at[idx_v], rf_v, sem).wait()
        pltpu.async_copy(hsb_hbm.at[idx_v], rb_v, sem).wait()
        pltpu.sync_copy(rf_v, updf_hbm.at[pl.ds(base, _ROWS_PER_W)])
        pltpu.sync_copy(rb_v, updb_hbm.at[pl.ds(base, _ROWS_PER_W)])


@functools.cache
def _gather_sc_kernel():
    return pl.kernel(
        _gather_sc_body,
        out_type=[
            jax.ShapeDtypeStruct((B, D), jnp.float32),
            jax.ShapeDtypeStruct((B, D), jnp.float32),
        ],
        mesh=plsc.VectorSubcoreMesh(core_axis_name="c", subcore_axis_name="s"),
        scratch_types=[
            pltpu.VMEM((_ROWS_PER_W,), jnp.int32),
            pltpu.VMEM((_ROWS_PER_W, D), jnp.float32),
            pltpu.VMEM((_ROWS_PER_W, D), jnp.float32),
            pltpu.SemaphoreType.DMA,
        ],
    )


def kernel(context_embeds, params, tokens):
    p = params

    wc_f = jnp.concatenate([p['Wch'][:, :D], p['Wcc'][:, :D]], axis=1)
    wc_b = jnp.concatenate([p['Wch'][:, D:], p['Wcc'][:, D:]], axis=1)
    b0_f = jnp.concatenate([p['bch'][:D], p['bcc'][:D]]).reshape(1, D2)
    b0_b = jnp.concatenate([p['bch'][D:], p['bcc'][D:]]).reshape(1, D2)
    names = ['mod', 'dele', 'ins', 'stop']
    w1 = jnp.concatenate([p[nm + '_W1'] for nm in names], axis=1)
    b1 = jnp.concatenate([p[nm + '_b1'] for nm in names]).reshape(1, NHEAD * D2)
    w2 = jnp.zeros((NHEAD * D2, NHEAD), jnp.float32)
    for k, nm in enumerate(names):
        w2 = w2.at[k * D2:(k + 1) * D2, k].set(p[nm + '_W2'][:, 0])
    b2 = jnp.stack([p[nm + '_b2'][0] for nm in names]).reshape(NHEAD, 1)

    tff, tfb, hcf, hcb = _prep(p['emb'], p['Wi_f'], p['b_f'].reshape(1, D4),
                               p['Wi_b'], p['b_b'].reshape(1, D4),
                               context_embeds, wc_f, b0_f, wc_b, b0_b)
    hs_f, hs_b = _lstm_scan(tokens, tff, p['Wh_f'], tfb, p['Wh_b'], hcf, hcb)
    ll, tgt, gidx = _heads(hs_f, hs_b, w1, b1, w2, b2, tokens)
    updf, updb = _gather_sc_kernel()(hs_f.reshape(L * B, D),
                                     hs_b.reshape(L * B, D), gidx.reshape(B))
    update_embed = jnp.concatenate([updf, updb], axis=1)
    return ll.reshape(B, 1), tgt.reshape(B), update_embed


# speedup vs baseline: 9.9510x; 1.5322x over previous
import functools

import jax
import jax.numpy as jnp
from jax import lax
from jax.experimental import pallas as pl
from jax.experimental.pallas import tpu as pltpu
from jax.experimental.pallas import tpu_sc as plsc

N_INF = -1e10
L, B, D, V = 512, 128, 128, 128
TOK_PAD, TOK_START, TOK_CONST, TOK_SUB, TOK_STOP = 0, 1, 2, 3, 4
D4 = 4 * D
D2 = 2 * D
NHEAD = 4
TL = 8


def _sig(x):
    return 0.5 * jnp.tanh(0.5 * x) + 0.5


def _prep_body(emb_ref, wif_ref, bif_ref, wib_ref, bib_ref, ctx_ref,
               wcf_ref, b0f_ref, wcb_ref, b0b_ref,
               tff_ref, tfb_ref, hcf_ref, hcb_ref):
    emb = emb_ref[...]
    tff_ref[...] = jnp.dot(emb, wif_ref[...],
                           preferred_element_type=jnp.float32) + bif_ref[...]
    tfb_ref[...] = jnp.dot(emb, wib_ref[...],
                           preferred_element_type=jnp.float32) + bib_ref[...]
    ctx = ctx_ref[...]
    hcf_ref[...] = jnp.tanh(jnp.dot(ctx, wcf_ref[...],
                                    preferred_element_type=jnp.float32)
                            + b0f_ref[...])
    hcb_ref[...] = jnp.tanh(jnp.dot(ctx, wcb_ref[...],
                                    preferred_element_type=jnp.float32)
                            + b0b_ref[...])


def _prep(emb, wif, bif, wib, bib, ctx, wcf, b0f, wcb, b0b):
    return pl.pallas_call(
        _prep_body,
        out_shape=[
            jax.ShapeDtypeStruct((V, D4), jnp.float32),
            jax.ShapeDtypeStruct((V, D4), jnp.float32),
            jax.ShapeDtypeStruct((B, D2), jnp.float32),
            jax.ShapeDtypeStruct((B, D2), jnp.float32),
        ],
    )(emb, wif, bif, wib, bib, ctx, wcf, b0f, wcb, b0b)


SU = 8


def _cell(x, wh_ref, h, c):
    gates = x + jnp.dot(h, wh_ref[...], preferred_element_type=jnp.float32)
    i_g = _sig(gates[:, :D])
    f_g = _sig(gates[:, D:2 * D])
    g_g = jnp.tanh(gates[:, 2 * D:3 * D])
    o_g = _sig(gates[:, 3 * D:])
    c_n = f_g * c + i_g * g_g
    h_n = o_g * jnp.tanh(c_n)
    return h_n, c_n


def _scan_body(tokens_ref, tff_ref, whf_ref, tfb_ref, whb_ref,
               hcf_ref, hcb_ref, hsf_ref, hsb_ref,
               hf_s, cf_s, hb_s, cb_s):
    j = pl.program_id(0)

    @pl.when(j == 0)
    def _init():
        hf_s[...] = hcf_ref[:, :D]
        cf_s[...] = hcf_ref[:, D:]
        hb_s[...] = hcb_ref[:, :D]
        cb_s[...] = hcb_ref[:, D:]

    tok_f = tokens_ref[pl.ds(j * SU, SU), :].reshape(1, SU * B)
    tok_b = tokens_ref[pl.ds(L - (j + 1) * SU, SU), :].reshape(1, SU * B)
    iot = lax.broadcasted_iota(jnp.int32, (V, SU * B), 0)
    ohf = (iot == tok_f).astype(jnp.float32)
    ohb = (iot == tok_b).astype(jnp.float32)
    xf_all = lax.dot_general(ohf, tff_ref[...], (((0,), (0,)), ((), ())),
                             preferred_element_type=jnp.float32)
    xb_all = lax.dot_general(ohb, tfb_ref[...], (((0,), (0,)), ((), ())),
                             preferred_element_type=jnp.float32)

    hf, cf = hf_s[...], cf_s[...]
    hb, cb = hb_s[...], cb_s[...]
    for u in range(SU):
        hf, cf = _cell(xf_all[u * B:(u + 1) * B], whf_ref, hf, cf)
        hsf_ref[u, :, :] = hf
        hb, cb = _cell(xb_all[(SU - 1 - u) * B:(SU - u) * B], whb_ref, hb, cb)
        hsb_ref[SU - 1 - u, :, :] = hb
    hf_s[...] = hf
    cf_s[...] = cf
    hb_s[...] = hb
    cb_s[...] = cb


def _lstm_scan(tokens, tff, whf, tfb, whb, hcf, hcb):
    cparams = pltpu.CompilerParams(dimension_semantics=("arbitrary",))
    return pl.pallas_call(
        _scan_body,
        grid=(L // SU,),
        in_specs=[
            pl.BlockSpec((L, B), lambda j: (0, 0)),
            pl.BlockSpec((V, D4), lambda j: (0, 0)),
            pl.BlockSpec((D, D4), lambda j: (0, 0)),
            pl.BlockSpec((V, D4), lambda j: (0, 0)),
            pl.BlockSpec((D, D4), lambda j: (0, 0)),
            pl.BlockSpec((B, D2), lambda j: (0, 0)),
            pl.BlockSpec((B, D2), lambda j: (0, 0)),
        ],
        out_specs=[
            pl.BlockSpec((SU, B, D), lambda j: (j, 0, 0)),
            pl.BlockSpec((SU, B, D), lambda j: (L // SU - 1 - j, 0, 0)),
        ],
        out_shape=[
            jax.ShapeDtypeStruct((L, B, D), jnp.float32),
            jax.ShapeDtypeStruct((L, B, D), jnp.float32),
        ],
        scratch_shapes=[pltpu.VMEM((B, D), jnp.float32) for _ in range(4)],
        compiler_params=cparams,
    )(tokens, tff, whf, tfb, whb, hcf, hcb)


def _heads_body(hf_ref, hb_ref, w1_ref, b1_ref, w2_ref, b2_ref, tok_ref,
                ll_ref, tgt_ref, gidx_ref, sc_s):
    j = pl.program_id(0)
    hf = hf_ref[...].reshape(TL * B, D)
    hb = hb_ref[...].reshape(TL * B, D)
    out2 = jnp.concatenate([hf, hb], axis=1)
    hid = jnp.dot(out2, w1_ref[...], preferred_element_type=jnp.float32)
    hid = jnp.maximum(hid + b1_ref[...], 0.0)
    st = lax.dot_general(w2_ref[...], hid, (((0,), (1,)), ((), ())),
                         preferred_element_type=jnp.float32)
    st = st + b2_ref[...]
    sc_s[:, pl.ds(j * TL, TL), :] = st.reshape(NHEAD, TL, B)

    @pl.when(j == L // TL - 1)
    def _assemble():
        tok = tok_ref[...]
        mod_s = sc_s[0]
        del_s = sc_s[1]
        ins_s = sc_s[2]
        stop_s = sc_s[3]
        expr = (tok == TOK_CONST) | (tok == TOK_SUB)
        zf = jnp.zeros((1, B), dtype=jnp.float32)
        expr_f = expr.astype(jnp.float32)
        expr_sh = jnp.concatenate([zf, expr_f[:-1]], axis=0) != 0.0
        del_sh = jnp.concatenate([zf, del_s[:-1]], axis=0)
        score = jnp.full((L, B), N_INF, dtype=jnp.float32)
        score = jnp.where(expr, mod_s, score)
        score = jnp.where(expr_sh, del_sh, score)
        score = jnp.where(tok == TOK_START, ins_s, score)
        score = jnp.where(tok == TOK_STOP, stop_s, score)
        m = jnp.max(score, axis=0, keepdims=True)
        z = jnp.log(jnp.sum(jnp.exp(score - m), axis=0, keepdims=True))
        ll_ref[...] = -z
        iot = lax.broadcasted_iota(jnp.int32, (L, B), 0)
        cand = jnp.where(score == m, iot, L)
        tgt = jnp.min(cand, axis=0, keepdims=True)
        tgt_ref[...] = tgt
        gidx_ref[...] = tgt * B + lax.broadcasted_iota(jnp.int32, (1, B), 1)


def _heads(hs_f, hs_b, w1, b1, w2, b2, tokens):
    return pl.pallas_call(
        _heads_body,
        grid=(L // TL,),
        in_specs=[
            pl.BlockSpec((TL, B, D), lambda j: (j, 0, 0)),
            pl.BlockSpec((TL, B, D), lambda j: (j, 0, 0)),
            pl.BlockSpec((D2, NHEAD * D2), lambda j: (0, 0)),
            pl.BlockSpec((1, NHEAD * D2), lambda j: (0, 0)),
            pl.BlockSpec((NHEAD * D2, NHEAD), lambda j: (0, 0)),
            pl.BlockSpec((NHEAD, 1), lambda j: (0, 0)),
            pl.BlockSpec((L, B), lambda j: (0, 0)),
        ],
        out_specs=[
            pl.BlockSpec((1, B), lambda j: (0, 0)),
            pl.BlockSpec((1, B), lambda j: (0, 0)),
            pl.BlockSpec((1, B), lambda j: (0, 0)),
        ],
        out_shape=[
            jax.ShapeDtypeStruct((1, B), jnp.float32),
            jax.ShapeDtypeStruct((1, B), jnp.int32),
            jax.ShapeDtypeStruct((1, B), jnp.int32),
        ],
        scratch_shapes=[pltpu.VMEM((NHEAD, L, B), jnp.float32)],
        compiler_params=pltpu.CompilerParams(
            dimension_semantics=("arbitrary",)),
    )(hs_f, hs_b, w1, b1, w2, b2, tokens)


_ROWS_PER_W = 16
_NW_ACT = B // _ROWS_PER_W


def _gather_sc_body(hsf_hbm, hsb_hbm, gidx_hbm, upd_hbm,
                    idx_v, rf_v, rb_v, sem):
    wid = lax.axis_index("s") * 2 + lax.axis_index("c")

    @pl.when(wid < _NW_ACT)
    def _():
        base = wid * _ROWS_PER_W
        pltpu.sync_copy(gidx_hbm.at[pl.ds(base, _ROWS_PER_W)], idx_v)
        pltpu.async_copy(hsf_hbm.at[idx_v], rf_v, sem).wait()
        pltpu.async_copy(hsb_hbm.at[idx_v], rb_v, sem).wait()
        pltpu.sync_copy(rf_v, upd_hbm.at[pl.ds(base, _ROWS_PER_W), pl.ds(0, D)])
        pltpu.sync_copy(rb_v, upd_hbm.at[pl.ds(base, _ROWS_PER_W), pl.ds(D, D)])


@functools.cache
def _gather_sc_kernel():
    return pl.kernel(
        _gather_sc_body,
        out_type=jax.ShapeDtypeStruct((B, D2), jnp.float32),
        mesh=plsc.VectorSubcoreMesh(core_axis_name="c", subcore_axis_name="s"),
        scratch_types=[
            pltpu.VMEM((_ROWS_PER_W,), jnp.int32),
            pltpu.VMEM((_ROWS_PER_W, D), jnp.float32),
            pltpu.VMEM((_ROWS_PER_W, D), jnp.float32),
            pltpu.SemaphoreType.DMA,
        ],
    )


def kernel(context_embeds, params, tokens):
    p = params

    wc_f = jnp.concatenate([p['Wch'][:, :D], p['Wcc'][:, :D]], axis=1)
    wc_b = jnp.concatenate([p['Wch'][:, D:], p['Wcc'][:, D:]], axis=1)
    b0_f = jnp.concatenate([p['bch'][:D], p['bcc'][:D]]).reshape(1, D2)
    b0_b = jnp.concatenate([p['bch'][D:], p['bcc'][D:]]).reshape(1, D2)
    names = ['mod', 'dele', 'ins', 'stop']
    w1 = jnp.concatenate([p[nm + '_W1'] for nm in names], axis=1)
    b1 = jnp.concatenate([p[nm + '_b1'] for nm in names]).reshape(1, NHEAD * D2)
    w2 = jnp.zeros((NHEAD * D2, NHEAD), jnp.float32)
    for k, nm in enumerate(names):
        w2 = w2.at[k * D2:(k + 1) * D2, k].set(p[nm + '_W2'][:, 0])
    b2 = jnp.stack([p[nm + '_b2'][0] for nm in names]).reshape(NHEAD, 1)

    tff, tfb, hcf, hcb = _prep(p['emb'], p['Wi_f'], p['b_f'].reshape(1, D4),
                               p['Wi_b'], p['b_b'].reshape(1, D4),
                               context_embeds, wc_f, b0_f, wc_b, b0_b)
    hs_f, hs_b = _lstm_scan(tokens, tff, p['Wh_f'], tfb, p['Wh_b'], hcf, hcb)
    ll, tgt, gidx = _heads(hs_f, hs_b, w1, b1, w2, b2, tokens)
    update_embed = _gather_sc_kernel()(hs_f.reshape(L * B, D),
                                       hs_b.reshape(L * B, D), gidx.reshape(B))
    return ll.reshape(B, 1), tgt.reshape(B), update_embed


# speedup vs baseline: 10.1628x; 1.0213x over previous
import functools

import jax
import jax.numpy as jnp
from jax import lax
from jax.experimental import pallas as pl
from jax.experimental.pallas import tpu as pltpu
from jax.experimental.pallas import tpu_sc as plsc

N_INF = -1e10
L, B, D, V = 512, 128, 128, 128
TOK_PAD, TOK_START, TOK_CONST, TOK_SUB, TOK_STOP = 0, 1, 2, 3, 4
D4 = 4 * D
D2 = 2 * D
NHEAD = 4
TL = 8


def _sig(x):
    return 0.5 * jnp.tanh(0.5 * x) + 0.5


def _prep_body(emb_ref, wif_ref, bif_ref, wib_ref, bib_ref, ctx_ref,
               wcf_ref, b0f_ref, wcb_ref, b0b_ref,
               tff_ref, tfb_ref, hcf_ref, hcb_ref):
    emb = emb_ref[...]
    tff_ref[...] = jnp.dot(emb, wif_ref[...],
                           preferred_element_type=jnp.float32) + bif_ref[...]
    tfb_ref[...] = jnp.dot(emb, wib_ref[...],
                           preferred_element_type=jnp.float32) + bib_ref[...]
    ctx = ctx_ref[...]
    hcf_ref[...] = jnp.tanh(jnp.dot(ctx, wcf_ref[...],
                                    preferred_element_type=jnp.float32)
                            + b0f_ref[...])
    hcb_ref[...] = jnp.tanh(jnp.dot(ctx, wcb_ref[...],
                                    preferred_element_type=jnp.float32)
                            + b0b_ref[...])


def _prep(emb, wif, bif, wib, bib, ctx, wcf, b0f, wcb, b0b):
    return pl.pallas_call(
        _prep_body,
        out_shape=[
            jax.ShapeDtypeStruct((V, D4), jnp.float32),
            jax.ShapeDtypeStruct((V, D4), jnp.float32),
            jax.ShapeDtypeStruct((B, D2), jnp.float32),
            jax.ShapeDtypeStruct((B, D2), jnp.float32),
        ],
    )(emb, wif, bif, wib, bib, ctx, wcf, b0f, wcb, b0b)


SU = 8


def _cell(x, wh_ref, h, c):
    gates = x + jnp.dot(h, wh_ref[...], preferred_element_type=jnp.float32)
    i_g = _sig(gates[:, :D])
    f_g = _sig(gates[:, D:2 * D])
    g_g = jnp.tanh(gates[:, 2 * D:3 * D])
    o_g = _sig(gates[:, 3 * D:])
    c_n = f_g * c + i_g * g_g
    h_n = o_g * jnp.tanh(c_n)
    return h_n, c_n


def _scan_body(tokens_ref, tff_ref, whf_ref, tfb_ref, whb_ref,
               hcf_ref, hcb_ref, hsf_ref, hsb_ref,
               hf_s, cf_s, hb_s, cb_s):
    j = pl.program_id(0)

    @pl.when(j == 0)
    def _init():
        hf_s[...] = hcf_ref[:, :D]
        cf_s[...] = hcf_ref[:, D:]
        hb_s[...] = hcb_ref[:, :D]
        cb_s[...] = hcb_ref[:, D:]

    iot = lax.broadcasted_iota(jnp.int32, (V, B), 0)

    def _xlook(t, tf_ref):
        oh = (iot == tokens_ref[pl.ds(t, 1), :]).astype(jnp.float32)
        return lax.dot_general(oh, tf_ref[...], (((0,), (0,)), ((), ())),
                               preferred_element_type=jnp.float32)

    xfs = [_xlook(j * SU + u, tff_ref) for u in range(SU)]
    xbs = [_xlook(L - 1 - (j * SU + u), tfb_ref) for u in range(SU)]

    hf, cf = hf_s[...], cf_s[...]
    hb, cb = hb_s[...], cb_s[...]
    for u in range(SU):
        hf, cf = _cell(xfs[u], whf_ref, hf, cf)
        hsf_ref[u, :, :] = hf
        hb, cb = _cell(xbs[u], whb_ref, hb, cb)
        hsb_ref[SU - 1 - u, :, :] = hb
    hf_s[...] = hf
    cf_s[...] = cf
    hb_s[...] = hb
    cb_s[...] = cb


def _lstm_scan(tokens, tff, whf, tfb, whb, hcf, hcb):
    cparams = pltpu.CompilerParams(dimension_semantics=("arbitrary",))
    return pl.pallas_call(
        _scan_body,
        grid=(L // SU,),
        in_specs=[
            pl.BlockSpec((L, B), lambda j: (0, 0)),
            pl.BlockSpec((V, D4), lambda j: (0, 0)),
            pl.BlockSpec((D, D4), lambda j: (0, 0)),
            pl.BlockSpec((V, D4), lambda j: (0, 0)),
            pl.BlockSpec((D, D4), lambda j: (0, 0)),
            pl.BlockSpec((B, D2), lambda j: (0, 0)),
            pl.BlockSpec((B, D2), lambda j: (0, 0)),
        ],
        out_specs=[
            pl.BlockSpec((SU, B, D), lambda j: (j, 0, 0)),
            pl.BlockSpec((SU, B, D), lambda j: (L // SU - 1 - j, 0, 0)),
        ],
        out_shape=[
            jax.ShapeDtypeStruct((L, B, D), jnp.float32),
            jax.ShapeDtypeStruct((L, B, D), jnp.float32),
        ],
        scratch_shapes=[pltpu.VMEM((B, D), jnp.float32) for _ in range(4)],
        compiler_params=cparams,
    )(tokens, tff, whf, tfb, whb, hcf, hcb)


def _heads_body(hf_ref, hb_ref, w1_ref, b1_ref, w2_ref, b2_ref, tok_ref,
                ll_ref, tgt_ref, gidx_ref, sc_s):
    j = pl.program_id(0)
    hf = hf_ref[...].reshape(TL * B, D)
    hb = hb_ref[...].reshape(TL * B, D)
    out2 = jnp.concatenate([hf, hb], axis=1)
    hid = jnp.dot(out2, w1_ref[...], preferred_element_type=jnp.float32)
    hid = jnp.maximum(hid + b1_ref[...], 0.0)
    st = lax.dot_general(w2_ref[...], hid, (((0,), (1,)), ((), ())),
                         preferred_element_type=jnp.float32)
    st = st + b2_ref[...]
    sc_s[:, pl.ds(j * TL, TL), :] = st.reshape(NHEAD, TL, B)

    @pl.when(j == L // TL - 1)
    def _assemble():
        tok = tok_ref[...]
        mod_s = sc_s[0]
        del_s = sc_s[1]
        ins_s = sc_s[2]
        stop_s = sc_s[3]
        expr = (tok == TOK_CONST) | (tok == TOK_SUB)
        zf = jnp.zeros((1, B), dtype=jnp.float32)
        expr_f = expr.astype(jnp.float32)
        expr_sh = jnp.concatenate([zf, expr_f[:-1]], axis=0) != 0.0
        del_sh = jnp.concatenate([zf, del_s[:-1]], axis=0)
        score = jnp.full((L, B), N_INF, dtype=jnp.float32)
        score = jnp.where(expr, mod_s, score)
        score = jnp.where(expr_sh, del_sh, score)
        score = jnp.where(tok == TOK_START, ins_s, score)
        score = jnp.where(tok == TOK_STOP, stop_s, score)
        m = jnp.max(score, axis=0, keepdims=True)
        z = jnp.log(jnp.sum(jnp.exp(score - m), axis=0, keepdims=True))
        ll_ref[...] = -z
        iot = lax.broadcasted_iota(jnp.int32, (L, B), 0)
        cand = jnp.where(score == m, iot, L)
        tgt = jnp.min(cand, axis=0, keepdims=True)
        tgt_ref[...] = tgt
        gidx_ref[...] = tgt * B + lax.broadcasted_iota(jnp.int32, (1, B), 1)


def _heads(hs_f, hs_b, w1, b1, w2, b2, tokens):
    return pl.pallas_call(
        _heads_body,
        grid=(L // TL,),
        in_specs=[
            pl.BlockSpec((TL, B, D), lambda j: (j, 0, 0)),
            pl.BlockSpec((TL, B, D), lambda j: (j, 0, 0)),
            pl.BlockSpec((D2, NHEAD * D2), lambda j: (0, 0)),
            pl.BlockSpec((1, NHEAD * D2), lambda j: (0, 0)),
            pl.BlockSpec((NHEAD * D2, NHEAD), lambda j: (0, 0)),
            pl.BlockSpec((NHEAD, 1), lambda j: (0, 0)),
            pl.BlockSpec((L, B), lambda j: (0, 0)),
        ],
        out_specs=[
            pl.BlockSpec((1, B), lambda j: (0, 0)),
            pl.BlockSpec((1, B), lambda j: (0, 0)),
            pl.BlockSpec((1, B), lambda j: (0, 0)),
        ],
        out_shape=[
            jax.ShapeDtypeStruct((1, B), jnp.float32),
            jax.ShapeDtypeStruct((1, B), jnp.int32),
            jax.ShapeDtypeStruct((1, B), jnp.int32),
        ],
        scratch_shapes=[pltpu.VMEM((NHEAD, L, B), jnp.float32)],
        compiler_params=pltpu.CompilerParams(
            dimension_semantics=("arbitrary",)),
    )(hs_f, hs_b, w1, b1, w2, b2, tokens)


_ROWS_PER_W = 16
_NW_ACT = B // _ROWS_PER_W


def _gather_sc_body(hsf_hbm, hsb_hbm, gidx_hbm, updf_hbm, updb_hbm,
                    idx_v, rf_v, rb_v, sem):
    wid = lax.axis_index("s") * 2 + lax.axis_index("c")

    @pl.when(wid < _NW_ACT)
    def _():
        base = wid * _ROWS_PER_W
        pltpu.sync_copy(gidx_hbm.at[pl.ds(base, _ROWS_PER_W)], idx_v)
        pltpu.async_copy(hsf_hbm.at[idx_v], rf_v, sem).wait()
        pltpu.async_copy(hsb_hbm.at[idx_v], rb_v, sem).wait()
        pltpu.sync_copy(rf_v, updf_hbm.at[pl.ds(base, _ROWS_PER_W)])
        pltpu.sync_copy(rb_v, updb_hbm.at[pl.ds(base, _ROWS_PER_W)])


@functools.cache
def _gather_sc_kernel():
    return pl.kernel(
        _gather_sc_body,
        out_type=[
            jax.ShapeDtypeStruct((B, D), jnp.float32),
            jax.ShapeDtypeStruct((B, D), jnp.float32),
        ],
        mesh=plsc.VectorSubcoreMesh(core_axis_name="c", subcore_axis_name="s"),
        scratch_types=[
            pltpu.VMEM((_ROWS_PER_W,), jnp.int32),
            pltpu.VMEM((_ROWS_PER_W, D), jnp.float32),
            pltpu.VMEM((_ROWS_PER_W, D), jnp.float32),
            pltpu.SemaphoreType.DMA,
        ],
    )


def kernel(context_embeds, params, tokens):
    p = params

    wc_f = jnp.concatenate([p['Wch'][:, :D], p['Wcc'][:, :D]], axis=1)
    wc_b = jnp.concatenate([p['Wch'][:, D:], p['Wcc'][:, D:]], axis=1)
    b0_f = jnp.concatenate([p['bch'][:D], p['bcc'][:D]]).reshape(1, D2)
    b0_b = jnp.concatenate([p['bch'][D:], p['bcc'][D:]]).reshape(1, D2)
    names = ['mod', 'dele', 'ins', 'stop']
    w1 = jnp.concatenate([p[nm + '_W1'] for nm in names], axis=1)
    b1 = jnp.concatenate([p[nm + '_b1'] for nm in names]).reshape(1, NHEAD * D2)
    w2 = jnp.zeros((NHEAD * D2, NHEAD), jnp.float32)
    for k, nm in enumerate(names):
        w2 = w2.at[k * D2:(k + 1) * D2, k].set(p[nm + '_W2'][:, 0])
    b2 = jnp.stack([p[nm + '_b2'][0] for nm in names]).reshape(NHEAD, 1)

    tff, tfb, hcf, hcb = _prep(p['emb'], p['Wi_f'], p['b_f'].reshape(1, D4),
                               p['Wi_b'], p['b_b'].reshape(1, D4),
                               context_embeds, wc_f, b0_f, wc_b, b0_b)
    hs_f, hs_b = _lstm_scan(tokens, tff, p['Wh_f'], tfb, p['Wh_b'], hcf, hcb)
    ll, tgt, gidx = _heads(hs_f, hs_b, w1, b1, w2, b2, tokens)
    updf, updb = _gather_sc_kernel()(hs_f.reshape(L * B, D),
                                     hs_b.reshape(L * B, D), gidx.reshape(B))
    update_embed = jnp.concatenate([updf, updb], axis=1)
    return ll.reshape(B, 1), tgt.reshape(B), update_embed
